# baseline (speedup 1.0000x reference)
import functools

import jax
import jax.numpy as jnp
from jax import lax
from jax.experimental import pallas as pl
from jax.experimental.pallas import tpu as pltpu
from jax.experimental.pallas import tpu_sc as plsc

_ROW_BLK = 2000
_CHUNK = 256
_LANES = 16
_NWORKERS = 32


def _proj_body(wr_ref, wi_ref, pr_ref, pi_ref, br_ref, bi_ref,
               tr_ref, ti_ref, mx_ref):
    wr = wr_ref[...].astype(jnp.bfloat16).astype(jnp.float32)
    wi = wi_ref[...].astype(jnp.bfloat16).astype(jnp.float32)
    p_r = pr_ref[...].astype(jnp.bfloat16).astype(jnp.float32)
    p_i = pi_ref[...].astype(jnp.bfloat16).astype(jnp.float32)
    tr = jnp.sum(wr * p_r - wi * p_i, axis=1, keepdims=True) + br_ref[0, 0]
    ti = jnp.sum(wr * p_i + wi * p_r, axis=1, keepdims=True) + bi_ref[0, 0]
    tr_ref[...] = tr
    ti_ref[...] = ti
    bm = jnp.maximum(jnp.max(jnp.abs(tr)), jnp.max(jnp.abs(ti)))
    bm = bm.reshape(1, 1)

    @pl.when(pl.program_id(0) == 0)
    def _():
        mx_ref[...] = bm

    @pl.when(pl.program_id(0) > 0)
    def _():
        mx_ref[...] = jnp.maximum(mx_ref[...], bm)


def _pack_body(tr_ref, ti_ref, sc_ref, out_ref):
    s = sc_ref[0, 0]
    yr = tr_ref[...] * s
    yi = ti_ref[...] * s
    yr = yr + jnp.where(yr >= 0, 0.5, -0.5)
    yi = yi + jnp.where(yi >= 0, 0.5, -0.5)
    ri = yr.astype(jnp.int32)
    ii = yi.astype(jnp.int32)
    out_ref[...] = jnp.bitwise_or(
        lax.shift_left(ii, 16), jnp.bitwise_and(ri, jnp.int32(0xFFFF)))


def _project_pack(Wh_real, Wh_imag, W_real, W_imag, b_real, b_imag):
    m, d = Wh_real.shape
    grid = m // _ROW_BLK
    tr, ti, mx = pl.pallas_call(
        _proj_body,
        grid=(grid,),
        in_specs=[
            pl.BlockSpec((_ROW_BLK, d), lambda i: (i, 0)),
            pl.BlockSpec((_ROW_BLK, d), lambda i: (i, 0)),
            pl.BlockSpec((1, d), lambda i: (0, 0)),
            pl.BlockSpec((1, d), lambda i: (0, 0)),
            pl.BlockSpec((1, 1), lambda i: (0, 0)),
            pl.BlockSpec((1, 1), lambda i: (0, 0)),
        ],
        out_specs=[
            pl.BlockSpec((_ROW_BLK, 1), lambda i: (i, 0)),
            pl.BlockSpec((_ROW_BLK, 1), lambda i: (i, 0)),
            pl.BlockSpec((1, 1), lambda i: (0, 0)),
        ],
        out_shape=[
            jax.ShapeDtypeStruct((m, 1), jnp.float32),
            jax.ShapeDtypeStruct((m, 1), jnp.float32),
            jax.ShapeDtypeStruct((1, 1), jnp.float32),
        ],
    )(Wh_real, Wh_imag, W_real, W_imag,
      b_real.reshape(1, 1), b_imag.reshape(1, 1))
    scale = 32767.0 / jnp.maximum(mx, jnp.float32(1e-30))
    packed = pl.pallas_call(
        _pack_body,
        grid=(grid,),
        in_specs=[
            pl.BlockSpec((_ROW_BLK, 1), lambda i: (i, 0)),
            pl.BlockSpec((_ROW_BLK, 1), lambda i: (i, 0)),
            pl.BlockSpec((1, 1), lambda i: (0, 0)),
        ],
        out_specs=pl.BlockSpec((_ROW_BLK, 1), lambda i: (i, 0)),
        out_shape=jax.ShapeDtypeStruct((m, 1), jnp.int32),
    )(tr, ti, scale)
    eps = (0.001 * scale[0, 0] * scale[0, 0]) * jnp.ones((16,), jnp.float32)
    return packed.reshape(m), eps


def _unpack_ri(word):
    r = lax.shift_right_arithmetic(lax.shift_left(word, 16), 16)
    i = lax.shift_right_arithmetic(word, 16)
    return r.astype(jnp.float32), i.astype(jnp.float32)


@functools.lru_cache(maxsize=None)
def _make_att_kernel(m, k, n):
    n_full = n // _CHUNK
    tail = n - n_full * _CHUNK
    n_chunks = n_full + (1 if tail else 0)
    full_groups = _CHUNK // _LANES
    tail_groups = tail // _LANES
    mesh = plsc.VectorSubcoreMesh(core_axis_name="c", subcore_axis_name="s")

    @functools.partial(
        pl.kernel,
        out_type=jax.ShapeDtypeStruct((k, n), jnp.float32),
        mesh=mesh,
        scratch_types=[
            pltpu.VMEM((m,), jnp.int32),
            pltpu.VMEM((k + 1, _CHUNK), jnp.int32),
            pltpu.VMEM((k, _CHUNK), jnp.float32),
            pltpu.VMEM((16,), jnp.float32),
        ],
        compiler_params=pltpu.CompilerParams(
            use_tc_tiling_on_sc=False, needs_layout_passes=False),
    )
    def att(tbl_hbm, eps_hbm, nneg_hbm, out_hbm, tbl_v, idx_v, out_v, eps_v):
        wid = lax.axis_index("s") * 2 + lax.axis_index("c")
        pltpu.sync_copy(tbl_hbm, tbl_v)
        pltpu.sync_copy(eps_hbm, eps_v)
        eps = eps_v[...]
        my_chunks = (n_chunks - wid + _NWORKERS - 1) // _NWORKERS

        def chunk_body(t, carry):
            cid = wid + t * _NWORKERS
            col = pl.multiple_of(cid * _CHUNK, 128)
            is_tail = cid == n_full if tail else False

            @pl.when(jnp.logical_not(is_tail))
            def _():
                pltpu.sync_copy(nneg_hbm.at[:, pl.ds(col, _CHUNK)], idx_v)

            if tail:
                @pl.when(is_tail)
                def _():
                    pltpu.sync_copy(nneg_hbm.at[:, pl.ds(col, tail)],
                                    idx_v.at[:, pl.ds(0, tail)])

            def group_body(g, carry2):
                sl = pl.ds(g * _LANES, _LANES)
                cw = plsc.load_gather(tbl_v, [idx_v[0, sl]])
                cr, ci = _unpack_ri(cw)
                acc = eps
                atts = []
                for kk in range(k):
                    w = plsc.load_gather(tbl_v, [idx_v[kk + 1, sl]])
                    r, im = _unpack_ri(w)
                    a = jnp.maximum(cr * r + ci * im, 0.0)
                    acc = acc + a
                    atts.append(a)
                inv = 1.0 / acc
                for kk in range(k):
                    out_v[kk, sl] = atts[kk] * inv
                return carry2

            groups = jnp.where(is_tail, tail_groups, full_groups)
            lax.fori_loop(0, groups, group_body, 0)

            @pl.when(jnp.logical_not(is_tail))
            def _():
                pltpu.sync_copy(out_v, out_hbm.at[:, pl.ds(col, _CHUNK)])

            if tail:
                @pl.when(is_tail)
                def _():
                    pltpu.sync_copy(out_v.at[:, pl.ds(0, tail)],
                                    out_hbm.at[:, pl.ds(col, tail)])
            return carry

        lax.fori_loop(0, my_chunks, chunk_body, 0)

    return att


def kernel(Wh_real, Wh_imag, W_real, W_imag, b_real, b_imag, N_neg, k_neighbors):
    m, _ = Wh_real.shape
    kp1, n = N_neg.shape
    tbl, eps = _project_pack(Wh_real, Wh_imag, W_real, W_imag, b_real, b_imag)
    att = _make_att_kernel(m, kp1 - 1, n)
    return att(tbl, eps, N_neg)


# speedup vs baseline: 1.4326x; 1.4326x over previous
import functools

import jax
import jax.numpy as jnp
from jax import lax
from jax.experimental import pallas as pl
from jax.experimental.pallas import tpu as pltpu
from jax.experimental.pallas import tpu_sc as plsc

_ROW_BLK = 2000
_CHUNK = 256
_LANES = 16
_NWORKERS = 32


def _proj_body(wr_ref, wi_ref, ws_ref, br_ref, bi_ref,
               tr_ref, ti_ref, mx_ref):
    wr = wr_ref[...].astype(jnp.bfloat16)
    wi = wi_ref[...].astype(jnp.bfloat16)
    ws = ws_ref[...]
    dn = (((1,), (1,)), ((), ()))
    pr = lax.dot_general(ws, wr, dn, preferred_element_type=jnp.float32)
    pi = lax.dot_general(ws, wi, dn, preferred_element_type=jnp.float32)
    tr = pr[0:1, :] - pi[1:2, :] + br_ref[0, 0]
    ti = pr[1:2, :] + pi[0:1, :] + bi_ref[0, 0]
    tr_ref[...] = tr.reshape(tr_ref.shape)
    ti_ref[...] = ti.reshape(ti_ref.shape)
    bm = jnp.maximum(jnp.max(jnp.abs(tr)), jnp.max(jnp.abs(ti)))
    bm = bm.reshape(1, 1)

    @pl.when(pl.program_id(0) == 0)
    def _():
        mx_ref[...] = bm

    @pl.when(pl.program_id(0) > 0)
    def _():
        mx_ref[...] = jnp.maximum(mx_ref[...], bm)


def _pack_body(tr_ref, ti_ref, sc_ref, out_ref):
    s = sc_ref[0, 0]
    yr = tr_ref[...] * s
    yi = ti_ref[...] * s
    yr = yr + jnp.where(yr >= 0, 0.5, -0.5)
    yi = yi + jnp.where(yi >= 0, 0.5, -0.5)
    ri = yr.astype(jnp.int32)
    ii = yi.astype(jnp.int32)
    out_ref[...] = jnp.bitwise_or(
        lax.shift_left(ii, 16), jnp.bitwise_and(ri, jnp.int32(0xFFFF)))


def _project_pack(Wh_real, Wh_imag, W_real, W_imag, b_real, b_imag):
    m, d = Wh_real.shape
    grid = m // _ROW_BLK
    w_stack = jnp.concatenate([W_real, W_imag], axis=0).astype(jnp.bfloat16)
    tr, ti, mx = pl.pallas_call(
        _proj_body,
        grid=(grid,),
        in_specs=[
            pl.BlockSpec((_ROW_BLK, d), lambda i: (i, 0)),
            pl.BlockSpec((_ROW_BLK, d), lambda i: (i, 0)),
            pl.BlockSpec((2, d), lambda i: (0, 0)),
            pl.BlockSpec((1, 1), lambda i: (0, 0)),
            pl.BlockSpec((1, 1), lambda i: (0, 0)),
        ],
        out_specs=[
            pl.BlockSpec((1, 1, _ROW_BLK), lambda i: (i, 0, 0)),
            pl.BlockSpec((1, 1, _ROW_BLK), lambda i: (i, 0, 0)),
            pl.BlockSpec((1, 1), lambda i: (0, 0)),
        ],
        out_shape=[
            jax.ShapeDtypeStruct((grid, 1, _ROW_BLK), jnp.float32),
            jax.ShapeDtypeStruct((grid, 1, _ROW_BLK), jnp.float32),
            jax.ShapeDtypeStruct((1, 1), jnp.float32),
        ],
    )(Wh_real, Wh_imag, w_stack,
      b_real.reshape(1, 1), b_imag.reshape(1, 1))
    scale = 32767.0 / jnp.maximum(mx, jnp.float32(1e-30))
    packed = pl.pallas_call(
        _pack_body,
        grid=(grid,),
        in_specs=[
            pl.BlockSpec((1, 1, _ROW_BLK), lambda i: (i, 0, 0)),
            pl.BlockSpec((1, 1, _ROW_BLK), lambda i: (i, 0, 0)),
            pl.BlockSpec((1, 1), lambda i: (0, 0)),
        ],
        out_specs=pl.BlockSpec((1, 1, _ROW_BLK), lambda i: (i, 0, 0)),
        out_shape=jax.ShapeDtypeStruct((grid, 1, _ROW_BLK), jnp.int32),
    )(tr, ti, scale)
    eps = (0.001 * scale[0, 0] * scale[0, 0]) * jnp.ones((16,), jnp.float32)
    return packed.reshape(m), eps


def _unpack_ri(word):
    r = lax.shift_right_arithmetic(lax.shift_left(word, 16), 16)
    i = lax.shift_right_arithmetic(word, 16)
    return r.astype(jnp.float32), i.astype(jnp.float32)


@functools.lru_cache(maxsize=None)
def _make_att_kernel(m, k, n):
    n_full = n // _CHUNK
    tail = n - n_full * _CHUNK
    n_chunks = n_full + (1 if tail else 0)
    full_groups = _CHUNK // _LANES
    tail_groups = tail // _LANES
    mesh = plsc.VectorSubcoreMesh(core_axis_name="c", subcore_axis_name="s")

    @functools.partial(
        pl.kernel,
        out_type=jax.ShapeDtypeStruct((k, n), jnp.float32),
        mesh=mesh,
        scratch_types=[
            pltpu.VMEM((m,), jnp.int32),
            pltpu.VMEM((k + 1, _CHUNK), jnp.int32),
            pltpu.VMEM((k, _CHUNK), jnp.float32),
            pltpu.VMEM((16,), jnp.float32),
        ],
        compiler_params=pltpu.CompilerParams(
            use_tc_tiling_on_sc=False, needs_layout_passes=False),
    )
    def att(tbl_hbm, eps_hbm, nneg_hbm, out_hbm, tbl_v, idx_v, out_v, eps_v):
        wid = lax.axis_index("s") * 2 + lax.axis_index("c")
        pltpu.sync_copy(tbl_hbm, tbl_v)
        pltpu.sync_copy(eps_hbm, eps_v)
        eps = eps_v[...]
        my_chunks = (n_chunks - wid + _NWORKERS - 1) // _NWORKERS

        def chunk_body(t, carry):
            cid = wid + t * _NWORKERS
            col = pl.multiple_of(cid * _CHUNK, 128)
            is_tail = cid == n_full if tail else False

            @pl.when(jnp.logical_not(is_tail))
            def _():
                pltpu.sync_copy(nneg_hbm.at[:, pl.ds(col, _CHUNK)], idx_v)

            if tail:
                @pl.when(is_tail)
                def _():
                    pltpu.sync_copy(nneg_hbm.at[:, pl.ds(col, tail)],
                                    idx_v.at[:, pl.ds(0, tail)])

            def group_body(g, carry2):
                sl = pl.ds(g * _LANES, _LANES)
                cw = plsc.load_gather(tbl_v, [idx_v[0, sl]])
                cr, ci = _unpack_ri(cw)
                acc = eps
                atts = []
                for kk in range(k):
                    w = plsc.load_gather(tbl_v, [idx_v[kk + 1, sl]])
                    r, im = _unpack_ri(w)
                    a = jnp.maximum(cr * r + ci * im, 0.0)
                    acc = acc + a
                    atts.append(a)
                inv = 1.0 / acc
                for kk in range(k):
                    out_v[kk, sl] = atts[kk] * inv
                return carry2

            groups = jnp.where(is_tail, tail_groups, full_groups)
            lax.fori_loop(0, groups, group_body, 0)

            @pl.when(jnp.logical_not(is_tail))
            def _():
                pltpu.sync_copy(out_v, out_hbm.at[:, pl.ds(col, _CHUNK)])

            if tail:
                @pl.when(is_tail)
                def _():
                    pltpu.sync_copy(out_v.at[:, pl.ds(0, tail)],
                                    out_hbm.at[:, pl.ds(col, tail)])
            return carry

        lax.fori_loop(0, my_chunks, chunk_body, 0)

    return att


def kernel(Wh_real, Wh_imag, W_real, W_imag, b_real, b_imag, N_neg, k_neighbors):
    m, _ = Wh_real.shape
    kp1, n = N_neg.shape
    tbl, eps = _project_pack(Wh_real, Wh_imag, W_real, W_imag, b_real, b_imag)
    att = _make_att_kernel(m, kp1 - 1, n)
    return att(tbl, eps, N_neg)


# speedup vs baseline: 2.3363x; 1.6308x over previous
import functools

import jax
import jax.numpy as jnp
from jax import lax
from jax.experimental import pallas as pl
from jax.experimental.pallas import tpu as pltpu
from jax.experimental.pallas import tpu_sc as plsc

_ROW_BLK = 5000
_CHUNK = 256
_LANES = 16
_NWORKERS = 32


def _proj_body(wr_ref, wi_ref, ws_ref, br_ref, bi_ref,
               tr_ref, ti_ref, mx_ref):
    wr = wr_ref[...].astype(jnp.bfloat16)
    wi = wi_ref[...].astype(jnp.bfloat16)
    ws = ws_ref[...]
    dn = (((1,), (1,)), ((), ()))
    pr = lax.dot_general(ws, wr, dn, preferred_element_type=jnp.float32)
    pi = lax.dot_general(ws, wi, dn, preferred_element_type=jnp.float32)
    tr = pr[0:1, :] - pi[1:2, :] + br_ref[0, 0]
    ti = pr[1:2, :] + pi[0:1, :] + bi_ref[0, 0]
    tr_ref[...] = tr.reshape(tr_ref.shape)
    ti_ref[...] = ti.reshape(ti_ref.shape)
    bm = jnp.maximum(jnp.max(jnp.abs(tr)), jnp.max(jnp.abs(ti)))
    bm = bm.reshape(1, 1)

    @pl.when(pl.program_id(0) == 0)
    def _():
        mx_ref[...] = bm

    @pl.when(pl.program_id(0) > 0)
    def _():
        mx_ref[...] = jnp.maximum(mx_ref[...], bm)


def _pack_body(tr_ref, ti_ref, sc_ref, out_ref):
    s = sc_ref[0, 0]
    yr = tr_ref[...] * s
    yi = ti_ref[...] * s
    yr = yr + jnp.where(yr >= 0, 0.5, -0.5)
    yi = yi + jnp.where(yi >= 0, 0.5, -0.5)
    ri = yr.astype(jnp.int32)
    ii = yi.astype(jnp.int32)
    out_ref[...] = jnp.bitwise_or(
        lax.shift_left(ii, 16), jnp.bitwise_and(ri, jnp.int32(0xFFFF)))


def _project_pack(Wh_real, Wh_imag, W_real, W_imag, b_real, b_imag):
    m, d = Wh_real.shape
    grid = m // _ROW_BLK
    w_stack = jnp.concatenate([W_real, W_imag], axis=0).astype(jnp.bfloat16)
    tr, ti, mx = pl.pallas_call(
        _proj_body,
        grid=(grid,),
        in_specs=[
            pl.BlockSpec((_ROW_BLK, d), lambda i: (i, 0)),
            pl.BlockSpec((_ROW_BLK, d), lambda i: (i, 0)),
            pl.BlockSpec((2, d), lambda i: (0, 0)),
            pl.BlockSpec((1, 1), lambda i: (0, 0)),
            pl.BlockSpec((1, 1), lambda i: (0, 0)),
        ],
        out_specs=[
            pl.BlockSpec((1, 1, _ROW_BLK), lambda i: (i, 0, 0)),
            pl.BlockSpec((1, 1, _ROW_BLK), lambda i: (i, 0, 0)),
            pl.BlockSpec((1, 1), lambda i: (0, 0)),
        ],
        out_shape=[
            jax.ShapeDtypeStruct((grid, 1, _ROW_BLK), jnp.float32),
            jax.ShapeDtypeStruct((grid, 1, _ROW_BLK), jnp.float32),
            jax.ShapeDtypeStruct((1, 1), jnp.float32),
        ],
    )(Wh_real, Wh_imag, w_stack,
      b_real.reshape(1, 1), b_imag.reshape(1, 1))
    scale = 32767.0 / jnp.maximum(mx, jnp.float32(1e-30))
    packed = pl.pallas_call(
        _pack_body,
        out_shape=jax.ShapeDtypeStruct((grid, 1, _ROW_BLK), jnp.int32),
    )(tr, ti, scale)
    eps = (0.001 * scale[0, 0] * scale[0, 0]) * jnp.ones((16,), jnp.float32)
    return packed.reshape(m), eps


def _unpack_ri(word):
    r = lax.shift_right_arithmetic(lax.shift_left(word, 16), 16)
    i = lax.shift_right_arithmetic(word, 16)
    return r.astype(jnp.float32), i.astype(jnp.float32)


@functools.lru_cache(maxsize=None)
def _make_att_kernel(m, k, n):
    n_full = n // _CHUNK
    tail = n - n_full * _CHUNK
    tail_main = (tail // 128) * 128
    tail_rem = tail % 128
    n_chunks = n_full + (1 if tail else 0)
    groups = _CHUNK // _LANES
    mesh = plsc.VectorSubcoreMesh(core_axis_name="c", subcore_axis_name="s")
    out_types = [jax.ShapeDtypeStruct((k, n), jnp.float32)]
    if tail_rem:
        out_types.append(jax.ShapeDtypeStruct((k, tail_rem), jnp.float32))

    @functools.partial(
        pl.kernel,
        out_type=out_types,
        mesh=mesh,
        scratch_types=[
            pltpu.VMEM((m,), jnp.int32),
            pltpu.VMEM((k + 1, _CHUNK), jnp.int32),
            pltpu.VMEM((k, _CHUNK), jnp.float32),
            pltpu.VMEM((16,), jnp.float32),
            pltpu.VMEM((k + 1, max(tail_rem, _LANES)), jnp.int32),
            pltpu.VMEM((k, max(tail_rem, _LANES)), jnp.float32),
        ],
        compiler_params=pltpu.CompilerParams(needs_layout_passes=False),
    )
    def att(tbl_hbm, eps_hbm, nneg_hbm, nnegt_hbm, out_hbm, outt_hbm,
            tbl_v, idx_v, out_v, eps_v, idxt_v, outt_v):
        wid = lax.axis_index("s") * 2 + lax.axis_index("c")
        pltpu.sync_copy(tbl_hbm, tbl_v)
        pltpu.sync_copy(eps_hbm, eps_v)
        eps = eps_v[...]
        my_chunks = (n_chunks - wid + _NWORKERS - 1) // _NWORKERS

        def make_group_body(src_ref, dst_ref):
            def group_body(g, carry2):
                sl = pl.ds(g * _LANES, _LANES)
                cw = plsc.load_gather(tbl_v, [src_ref[0, sl]])
                cr, ci = _unpack_ri(cw)
                acc = eps
                atts = []
                for kk in range(k):
                    w = plsc.load_gather(tbl_v, [src_ref[kk + 1, sl]])
                    r, im = _unpack_ri(w)
                    a = jnp.maximum(cr * r + ci * im, 0.0)
                    acc = acc + a
                    atts.append(a)
                inv = 1.0 / acc
                for kk in range(k):
                    dst_ref[kk, sl] = atts[kk] * inv
                return carry2
            return group_body

        def chunk_body(t, carry):
            cid = wid + t * _NWORKERS
            col = pl.multiple_of(cid * _CHUNK, 128)
            is_tail = cid == n_full if tail else False

            @pl.when(jnp.logical_not(is_tail))
            def _():
                pltpu.sync_copy(nneg_hbm.at[:, pl.ds(col, _CHUNK)], idx_v)
                lax.fori_loop(0, groups, make_group_body(idx_v, out_v), 0)
                pltpu.sync_copy(out_v, out_hbm.at[:, pl.ds(col, _CHUNK)])

            if tail:
                @pl.when(is_tail)
                def _():
                    if tail_main:
                        pltpu.sync_copy(
                            nneg_hbm.at[:, pl.ds(col, tail_main)],
                            idx_v.at[:, pl.ds(0, tail_main)])
                        lax.fori_loop(0, tail_main // _LANES,
                                      make_group_body(idx_v, out_v), 0)
                        pltpu.sync_copy(
                            out_v.at[:, pl.ds(0, tail_main)],
                            out_hbm.at[:, pl.ds(col, tail_main)])
                    if tail_rem:
                        pltpu.sync_copy(nnegt_hbm,
                                        idxt_v.at[:, pl.ds(0, tail_rem)])
                        lax.fori_loop(0, tail_rem // _LANES,
                                      make_group_body(idxt_v, outt_v), 0)
                        pltpu.sync_copy(outt_v.at[:, pl.ds(0, tail_rem)],
                                        outt_hbm)
            return carry

        lax.fori_loop(0, my_chunks, chunk_body, 0)

    return att


def kernel(Wh_real, Wh_imag, W_real, W_imag, b_real, b_imag, N_neg, k_neighbors):
    m, _ = Wh_real.shape
    kp1, n = N_neg.shape
    tbl, eps = _project_pack(Wh_real, Wh_imag, W_real, W_imag, b_real, b_imag)
    att = _make_att_kernel(m, kp1 - 1, n)
    tail_rem = n % 128
    nneg_t = lax.slice(N_neg, (0, n - tail_rem), (kp1, n))
    out, out_t = att(tbl, eps, N_neg, nneg_t)
    return lax.dynamic_update_slice(out, out_t, (0, n - tail_rem))


# speedup vs baseline: 2.6779x; 1.1462x over previous
import functools

import jax
import jax.numpy as jnp
from jax import lax
from jax.experimental import pallas as pl
from jax.experimental.pallas import tpu as pltpu
from jax.experimental.pallas import tpu_sc as plsc

_ROW_BLK = 5000
_CHUNK = 128
_LANES = 16
_NWORKERS = 32


def _proj_body(wr_ref, wi_ref, ws_ref, br_ref, bi_ref,
               tr_ref, ti_ref, mx_ref):
    wr = wr_ref[...].astype(jnp.bfloat16)
    wi = wi_ref[...].astype(jnp.bfloat16)
    ws = ws_ref[...]
    dn = (((1,), (1,)), ((), ()))
    pr = lax.dot_general(ws, wr, dn, preferred_element_type=jnp.float32)
    pi = lax.dot_general(ws, wi, dn, preferred_element_type=jnp.float32)
    tr = pr[0:1, :] - pi[1:2, :] + br_ref[0, 0]
    ti = pr[1:2, :] + pi[0:1, :] + bi_ref[0, 0]
    tr_ref[...] = tr.reshape(tr_ref.shape)
    ti_ref[...] = ti.reshape(ti_ref.shape)
    bm = jnp.maximum(jnp.max(jnp.abs(tr)), jnp.max(jnp.abs(ti)))
    bm = bm.reshape(1, 1)

    @pl.when(pl.program_id(0) == 0)
    def _():
        mx_ref[...] = bm

    @pl.when(pl.program_id(0) > 0)
    def _():
        mx_ref[...] = jnp.maximum(mx_ref[...], bm)


def _pack_body(tr_ref, ti_ref, sc_ref, out_ref):
    s = sc_ref[0, 0]
    yr = tr_ref[...] * s
    yi = ti_ref[...] * s
    yr = yr + jnp.where(yr >= 0, 0.5, -0.5)
    yi = yi + jnp.where(yi >= 0, 0.5, -0.5)
    ri = yr.astype(jnp.int32)
    ii = yi.astype(jnp.int32)
    out_ref[...] = jnp.bitwise_or(
        lax.shift_left(ii, 16), jnp.bitwise_and(ri, jnp.int32(0xFFFF)))


def _project_pack(Wh_real, Wh_imag, W_real, W_imag, b_real, b_imag):
    m, d = Wh_real.shape
    grid = m // _ROW_BLK
    w_stack = jnp.concatenate([W_real, W_imag], axis=0).astype(jnp.bfloat16)
    tr, ti, mx = pl.pallas_call(
        _proj_body,
        grid=(grid,),
        in_specs=[
            pl.BlockSpec((_ROW_BLK, d), lambda i: (i, 0)),
            pl.BlockSpec((_ROW_BLK, d), lambda i: (i, 0)),
            pl.BlockSpec((2, d), lambda i: (0, 0)),
            pl.BlockSpec((1, 1), lambda i: (0, 0)),
            pl.BlockSpec((1, 1), lambda i: (0, 0)),
        ],
        out_specs=[
            pl.BlockSpec((1, 1, _ROW_BLK), lambda i: (i, 0, 0)),
            pl.BlockSpec((1, 1, _ROW_BLK), lambda i: (i, 0, 0)),
            pl.BlockSpec((1, 1), lambda i: (0, 0)),
        ],
        out_shape=[
            jax.ShapeDtypeStruct((grid, 1, _ROW_BLK), jnp.float32),
            jax.ShapeDtypeStruct((grid, 1, _ROW_BLK), jnp.float32),
            jax.ShapeDtypeStruct((1, 1), jnp.float32),
        ],
    )(Wh_real, Wh_imag, w_stack,
      b_real.reshape(1, 1), b_imag.reshape(1, 1))
    scale = 32767.0 / jnp.maximum(mx, jnp.float32(1e-30))
    packed = pl.pallas_call(
        _pack_body,
        out_shape=jax.ShapeDtypeStruct((grid, 1, _ROW_BLK), jnp.int32),
    )(tr, ti, scale)
    eps = (0.001 * scale[0, 0] * scale[0, 0]) * jnp.ones((16,), jnp.float32)
    return packed.reshape(m), eps


def _unpack_ri(word):
    r = lax.shift_right_arithmetic(lax.shift_left(word, 16), 16)
    i = lax.shift_right_arithmetic(word, 16)
    return r.astype(jnp.float32), i.astype(jnp.float32)


@functools.lru_cache(maxsize=None)
def _make_att_kernel(m, k, n):
    n_full = n // _CHUNK
    tail = n - n_full * _CHUNK
    tail_main = (tail // 128) * 128
    tail_rem = tail % 128
    wid_tail = n_full % _NWORKERS
    col_tail = n_full * _CHUNK
    groups = _CHUNK // _LANES
    npairs = ((n_full + _NWORKERS - 1) // _NWORKERS + 1) // 2
    mesh = plsc.VectorSubcoreMesh(core_axis_name="c", subcore_axis_name="s")
    out_types = [jax.ShapeDtypeStruct((k, n), jnp.float32)]
    if tail_rem:
        out_types.append(jax.ShapeDtypeStruct((k, tail_rem), jnp.float32))

    @functools.partial(
        pl.kernel,
        out_type=out_types,
        mesh=mesh,
        scratch_types=[
            pltpu.VMEM((m,), jnp.int32),
            pltpu.VMEM((k + 1, _CHUNK), jnp.int32),
            pltpu.VMEM((k + 1, _CHUNK), jnp.int32),
            pltpu.VMEM((k, _CHUNK), jnp.float32),
            pltpu.VMEM((16,), jnp.float32),
            pltpu.VMEM((k + 1, max(tail_rem, _LANES)), jnp.int32),
            pltpu.VMEM((k, max(tail_rem, _LANES)), jnp.float32),
            pltpu.SemaphoreType.DMA,
            pltpu.SemaphoreType.DMA,
            pltpu.SemaphoreType.DMA,
            pltpu.SemaphoreType.DMA,
        ],
        compiler_params=pltpu.CompilerParams(needs_layout_passes=False),
    )
    def att(tbl_hbm, eps_hbm, nneg_hbm, nnegt_hbm, out_hbm, outt_hbm,
            tbl_v, idx0_v, idx1_v, out_v, eps_v, idxt_v, outt_v,
            sem_tbl, sem_in0, sem_in1, sem_out):
        wid = lax.axis_index("s") * 2 + lax.axis_index("c")
        pltpu.async_copy(tbl_hbm, tbl_v, sem_tbl)
        my_full = (n_full - wid + _NWORKERS - 1) // _NWORKERS

        def col_of(t):
            return pl.multiple_of((wid + t * _NWORKERS) * _CHUNK, 128)

        def start_idx(t, buf, sem):
            pltpu.async_copy(nneg_hbm.at[:, pl.ds(col_of(t), _CHUNK)],
                             buf, sem)

        def wait_idx(buf, sem):
            pltpu.make_async_copy(nneg_hbm.at[:, pl.ds(0, _CHUNK)],
                                  buf, sem).wait()

        def start_out(t):
            pltpu.async_copy(out_v, out_hbm.at[:, pl.ds(col_of(t), _CHUNK)],
                             sem_out)

        def wait_out():
            pltpu.make_async_copy(out_v, out_hbm.at[:, pl.ds(0, _CHUNK)],
                                  sem_out).wait()

        @pl.when(my_full > 0)
        def _():
            start_idx(0, idx0_v, sem_in0)
        pltpu.sync_copy(eps_hbm, eps_v)
        pltpu.make_async_copy(tbl_hbm, tbl_v, sem_tbl).wait()
        eps = eps_v[...]

        def make_group_body(src_ref, dst_ref):
            def group_body(g, carry2):
                sl = pl.ds(g * _LANES, _LANES)
                cw = plsc.load_gather(tbl_v, [src_ref[0, sl]])
                cr, ci = _unpack_ri(cw)
                acc = eps
                atts = []
                for kk in range(k):
                    w = plsc.load_gather(tbl_v, [src_ref[kk + 1, sl]])
                    r, im = _unpack_ri(w)
                    a = jnp.maximum(cr * r + ci * im, 0.0)
                    acc = acc + a
                    atts.append(a)
                inv = 1.0 / acc
                for kk in range(k):
                    dst_ref[kk, sl] = atts[kk] * inv
                return carry2
            return group_body

        def run_chunk(t, buf, sem, first):
            wait_idx(buf, sem)
            if not first:
                wait_out()
            lax.fori_loop(0, groups, make_group_body(buf, out_v), 0)
            start_out(t)

        def pair_body(tp, carry):
            t0 = 2 * tp
            t1 = t0 + 1

            @pl.when(t0 < my_full)
            def _():
                @pl.when(t1 < my_full)
                def _():
                    start_idx(t1, idx1_v, sem_in1)

                @pl.when(t0 == 0)
                def _():
                    run_chunk(t0, idx0_v, sem_in0, True)

                @pl.when(t0 > 0)
                def _():
                    run_chunk(t0, idx0_v, sem_in0, False)

            @pl.when(t1 < my_full)
            def _():
                @pl.when(t1 + 1 < my_full)
                def _():
                    start_idx(t1 + 1, idx0_v, sem_in0)
                run_chunk(t1, idx1_v, sem_in1, False)
            return carry

        lax.fori_loop(0, npairs, pair_body, 0)

        @pl.when(my_full > 0)
        def _():
            wait_out()

        if tail:
            @pl.when(wid == wid_tail)
            def _():
                if tail_main:
                    pltpu.sync_copy(
                        nneg_hbm.at[:, pl.ds(col_tail, tail_main)],
                        idx0_v.at[:, pl.ds(0, tail_main)])
                    lax.fori_loop(0, tail_main // _LANES,
                                  make_group_body(idx0_v, out_v), 0)
                    pltpu.sync_copy(
                        out_v.at[:, pl.ds(0, tail_main)],
                        out_hbm.at[:, pl.ds(col_tail, tail_main)])
                if tail_rem:
                    pltpu.sync_copy(nnegt_hbm,
                                    idxt_v.at[:, pl.ds(0, tail_rem)])
                    lax.fori_loop(0, tail_rem // _LANES,
                                  make_group_body(idxt_v, outt_v), 0)
                    pltpu.sync_copy(outt_v.at[:, pl.ds(0, tail_rem)],
                                    outt_hbm)

    return att


def kernel(Wh_real, Wh_imag, W_real, W_imag, b_real, b_imag, N_neg, k_neighbors):
    m, _ = Wh_real.shape
    kp1, n = N_neg.shape
    tbl, eps = _project_pack(Wh_real, Wh_imag, W_real, W_imag, b_real, b_imag)
    att = _make_att_kernel(m, kp1 - 1, n)
    tail_rem = n % 128
    nneg_t = lax.slice(N_neg, (0, n - tail_rem), (kp1, n))
    out, out_t = att(tbl, eps, N_neg, nneg_t)
    return lax.dynamic_update_slice(out, out_t, (0, n - tail_rem))


# speedup vs baseline: 2.8679x; 1.0709x over previous
import functools

import jax
import jax.numpy as jnp
from jax import lax
from jax.experimental import pallas as pl
from jax.experimental.pallas import tpu as pltpu
from jax.experimental.pallas import tpu_sc as plsc

_ROW_BLK = 10000
_CHUNK = 128
_LANES = 16
_NWORKERS = 32


def _proj_body(wr_ref, wi_ref, pr_w_ref, pi_w_ref, br_ref, bi_ref,
               tr_ref, ti_ref, mx_ref):
    wr = wr_ref[...].astype(jnp.bfloat16)
    wi = wi_ref[...].astype(jnp.bfloat16)
    ws = jnp.concatenate([pr_w_ref[...], pi_w_ref[...]],
                         axis=0).astype(jnp.bfloat16)
    dn = (((1,), (1,)), ((), ()))
    pr = lax.dot_general(ws, wr, dn, preferred_element_type=jnp.float32)
    pi = lax.dot_general(ws, wi, dn, preferred_element_type=jnp.float32)
    tr = pr[0:1, :] - pi[1:2, :] + br_ref[0, 0]
    ti = pr[1:2, :] + pi[0:1, :] + bi_ref[0, 0]
    tr_ref[...] = tr.reshape(tr_ref.shape)
    ti_ref[...] = ti.reshape(ti_ref.shape)
    bm = jnp.maximum(jnp.max(jnp.abs(tr)), jnp.max(jnp.abs(ti)))
    bm = bm.reshape(1, 1)

    @pl.when(pl.program_id(0) == 0)
    def _():
        mx_ref[...] = bm

    @pl.when(pl.program_id(0) > 0)
    def _():
        mx_ref[...] = jnp.maximum(mx_ref[...], bm)


def _pack_body(tr_ref, ti_ref, sc_ref, out_ref):
    s = sc_ref[0, 0]
    yr = tr_ref[...] * s
    yi = ti_ref[...] * s
    yr = yr + jnp.where(yr >= 0, 0.5, -0.5)
    yi = yi + jnp.where(yi >= 0, 0.5, -0.5)
    ri = yr.astype(jnp.int32)
    ii = yi.astype(jnp.int32)
    out_ref[...] = jnp.bitwise_or(
        lax.shift_left(ii, 16), jnp.bitwise_and(ri, jnp.int32(0xFFFF)))


def _project_pack(Wh_real, Wh_imag, W_real, W_imag, b_real, b_imag):
    m, d = Wh_real.shape
    grid = m // _ROW_BLK
    tr, ti, mx = pl.pallas_call(
        _proj_body,
        grid=(grid,),
        in_specs=[
            pl.BlockSpec((_ROW_BLK, d), lambda i: (i, 0)),
            pl.BlockSpec((_ROW_BLK, d), lambda i: (i, 0)),
            pl.BlockSpec((1, d), lambda i: (0, 0)),
            pl.BlockSpec((1, d), lambda i: (0, 0)),
            pl.BlockSpec((1, 1), lambda i: (0, 0)),
            pl.BlockSpec((1, 1), lambda i: (0, 0)),
        ],
        out_specs=[
            pl.BlockSpec((1, 1, _ROW_BLK), lambda i: (i, 0, 0)),
            pl.BlockSpec((1, 1, _ROW_BLK), lambda i: (i, 0, 0)),
            pl.BlockSpec((1, 1), lambda i: (0, 0)),
        ],
        out_shape=[
            jax.ShapeDtypeStruct((grid, 1, _ROW_BLK), jnp.float32),
            jax.ShapeDtypeStruct((grid, 1, _ROW_BLK), jnp.float32),
            jax.ShapeDtypeStruct((1, 1), jnp.float32),
        ],
    )(Wh_real, Wh_imag, W_real, W_imag,
      b_real.reshape(1, 1), b_imag.reshape(1, 1))
    scale = 32767.0 / jnp.maximum(mx, jnp.float32(1e-30))
    packed = pl.pallas_call(
        _pack_body,
        out_shape=jax.ShapeDtypeStruct((grid, 1, _ROW_BLK), jnp.int32),
    )(tr, ti, scale)
    eps = (0.001 * scale[0, 0] * scale[0, 0]) * jnp.ones((16,), jnp.float32)
    return packed.reshape(m), eps


def _unpack_ri(word):
    r = lax.shift_right_arithmetic(lax.shift_left(word, 16), 16)
    i = lax.shift_right_arithmetic(word, 16)
    return r.astype(jnp.float32), i.astype(jnp.float32)


@functools.lru_cache(maxsize=None)
def _make_att_kernel(m, k, n):
    n_full = n // _CHUNK
    tail = n - n_full * _CHUNK
    tail_main = (tail // 128) * 128
    tail_rem = tail % 128
    wid_tail = n_full % _NWORKERS
    col_tail = n_full * _CHUNK
    groups = _CHUNK // _LANES
    npairs = ((n_full + _NWORKERS - 1) // _NWORKERS + 1) // 2
    mesh = plsc.VectorSubcoreMesh(core_axis_name="c", subcore_axis_name="s")
    out_types = [jax.ShapeDtypeStruct((k, n), jnp.float32)]
    if tail_rem:
        out_types.append(jax.ShapeDtypeStruct((k, tail_rem), jnp.float32))

    @functools.partial(
        pl.kernel,
        out_type=out_types,
        mesh=mesh,
        scratch_types=[
            pltpu.VMEM((m,), jnp.int32),
            pltpu.VMEM((k + 1, _CHUNK), jnp.int32),
            pltpu.VMEM((k + 1, _CHUNK), jnp.int32),
            pltpu.VMEM((k, _CHUNK), jnp.float32),
            pltpu.VMEM((16,), jnp.float32),
            pltpu.VMEM((k + 1, max(tail_rem, _LANES)), jnp.int32),
            pltpu.VMEM((k, max(tail_rem, _LANES)), jnp.float32),
            pltpu.SemaphoreType.DMA,
            pltpu.SemaphoreType.DMA,
            pltpu.SemaphoreType.DMA,
            pltpu.SemaphoreType.DMA,
        ],
        compiler_params=pltpu.CompilerParams(needs_layout_passes=False),
    )
    def att(tbl_hbm, eps_hbm, nneg_hbm, nnegt_hbm, out_hbm, outt_hbm,
            tbl_v, idx0_v, idx1_v, out_v, eps_v, idxt_v, outt_v,
            sem_tbl, sem_in0, sem_in1, sem_out):
        wid = lax.axis_index("s") * 2 + lax.axis_index("c")
        pltpu.async_copy(tbl_hbm, tbl_v, sem_tbl)
        my_full = (n_full - wid + _NWORKERS - 1) // _NWORKERS

        def col_of(t):
            return pl.multiple_of((wid + t * _NWORKERS) * _CHUNK, 128)

        def start_idx(t, buf, sem):
            pltpu.async_copy(nneg_hbm.at[:, pl.ds(col_of(t), _CHUNK)],
                             buf, sem)

        def wait_idx(buf, sem):
            pltpu.make_async_copy(nneg_hbm.at[:, pl.ds(0, _CHUNK)],
                                  buf, sem).wait()

        def start_out(t):
            pltpu.async_copy(out_v, out_hbm.at[:, pl.ds(col_of(t), _CHUNK)],
                             sem_out)

        def wait_out():
            pltpu.make_async_copy(out_v, out_hbm.at[:, pl.ds(0, _CHUNK)],
                                  sem_out).wait()

        @pl.when(my_full > 0)
        def _():
            start_idx(0, idx0_v, sem_in0)
        pltpu.sync_copy(eps_hbm, eps_v)
        pltpu.make_async_copy(tbl_hbm, tbl_v, sem_tbl).wait()
        eps = eps_v[...]

        def make_group_body(src_ref, dst_ref):
            def group_body(g, carry2):
                sl = pl.ds(g * _LANES, _LANES)
                cw = plsc.load_gather(tbl_v, [src_ref[0, sl]])
                cr, ci = _unpack_ri(cw)
                acc = eps
                atts = []
                for kk in range(k):
                    w = plsc.load_gather(tbl_v, [src_ref[kk + 1, sl]])
                    r, im = _unpack_ri(w)
                    a = jnp.maximum(cr * r + ci * im, 0.0)
                    acc = acc + a
                    atts.append(a)
                inv = 1.0 / acc
                for kk in range(k):
                    dst_ref[kk, sl] = atts[kk] * inv
                return carry2
            return group_body

        def run_chunk(t, buf, sem, first):
            wait_idx(buf, sem)
            if not first:
                wait_out()
            lax.fori_loop(0, groups, make_group_body(buf, out_v), 0)
            start_out(t)

        def pair_body(tp, carry):
            t0 = 2 * tp
            t1 = t0 + 1

            @pl.when(t0 < my_full)
            def _():
                @pl.when(t1 < my_full)
                def _():
                    start_idx(t1, idx1_v, sem_in1)

                @pl.when(t0 == 0)
                def _():
                    run_chunk(t0, idx0_v, sem_in0, True)

                @pl.when(t0 > 0)
                def _():
                    run_chunk(t0, idx0_v, sem_in0, False)

            @pl.when(t1 < my_full)
            def _():
                @pl.when(t1 + 1 < my_full)
                def _():
                    start_idx(t1 + 1, idx0_v, sem_in0)
                run_chunk(t1, idx1_v, sem_in1, False)
            return carry

        lax.fori_loop(0, npairs, pair_body, 0)

        @pl.when(my_full > 0)
        def _():
            wait_out()

        if tail:
            @pl.when(wid == wid_tail)
            def _():
                if tail_main:
                    pltpu.sync_copy(
                        nneg_hbm.at[:, pl.ds(col_tail, tail_main)],
                        idx0_v.at[:, pl.ds(0, tail_main)])
                    lax.fori_loop(0, tail_main // _LANES,
                                  make_group_body(idx0_v, out_v), 0)
                    pltpu.sync_copy(
                        out_v.at[:, pl.ds(0, tail_main)],
                        out_hbm.at[:, pl.ds(col_tail, tail_main)])
                if tail_rem:
                    pltpu.sync_copy(nnegt_hbm,
                                    idxt_v.at[:, pl.ds(0, tail_rem)])
                    lax.fori_loop(0, tail_rem // _LANES,
                                  make_group_body(idxt_v, outt_v), 0)
                    pltpu.sync_copy(outt_v.at[:, pl.ds(0, tail_rem)],
                                    outt_hbm)

    return att


def kernel(Wh_real, Wh_imag, W_real, W_imag, b_real, b_imag, N_neg, k_neighbors):
    m, _ = Wh_real.shape
    kp1, n = N_neg.shape
    tbl, eps = _project_pack(Wh_real, Wh_imag, W_real, W_imag, b_real, b_imag)
    att = _make_att_kernel(m, kp1 - 1, n)
    tail_rem = n % 128
    nneg_t = lax.slice(N_neg, (0, n - tail_rem), (kp1, n))
    out, out_t = att(tbl, eps, N_neg, nneg_t)
    return lax.dynamic_update_slice(out, out_t, (0, n - tail_rem))
